# baseline (speedup 1.0000x reference)
import jax
import jax.numpy as jnp
from jax.experimental import pallas as pl
from jax.experimental.pallas import tpu as pltpu

_BM = 1024
_KA = 32


def _body(x_ref, w1_ref, b1_ref, w2_ref, b2_ref, bank_ref,
          out_ref, aug_ref, pTa_ref, psq_ref, acc_ref, tb_ref):
    i = pl.program_id(0)
    nm = pl.num_programs(0)
    n = x_ref.shape[0]

    @pl.when(i == 0)
    def _init():
        hT = jnp.maximum(
            jax.lax.dot_general(
                w1_ref[...], x_ref[...], (((0,), (1,)), ((), ())),
                preferred_element_type=jnp.float32) + b1_ref[...], 0.0)
        pT = jax.lax.dot_general(
            w2_ref[...], hT, (((0,), (0,)), ((), ())),
            preferred_element_type=jnp.float32) + b2_ref[...]
        psq_ref[...] = jnp.sum(pT * pT, axis=0, keepdims=True)
        d2 = pT.shape[0]
        pTa_ref[...] = jnp.concatenate(
            [-2.0 * pT,
             jnp.ones((1, n), jnp.float32),
             jnp.zeros((_KA - d2 - 1, n), jnp.float32)],
            axis=0).astype(jnp.bfloat16)
        bank = bank_ref[...]
        m = bank.shape[0]
        m_sq = jnp.sum(bank * bank, axis=1, keepdims=True)
        aug_ref[pl.ds(0, m), :] = jnp.concatenate(
            [bank, m_sq, jnp.zeros((m, _KA - d2 - 1), jnp.float32)],
            axis=1).astype(jnp.bfloat16)
        npad = aug_ref.shape[0] - m
        pad_rows = jnp.where(
            jax.lax.broadcasted_iota(jnp.int32, (npad, _KA), 1) == d2,
            jnp.float32(1e9), jnp.float32(0.0)).astype(jnp.bfloat16)
        aug_ref[pl.ds(m, npad), :] = pad_rows
        acc_ref[...] = jnp.full_like(acc_ref[...], jnp.inf)

        tb_ref[pl.ds(1, 1), :, :] = jnp.full(
            (1,) + tb_ref.shape[1:], jnp.inf, jnp.float32)

    par = jax.lax.rem(i, 2)
    nt = nm - 1
    tile_i = jnp.minimum(i, nt - 1)
    tile = aug_ref[pl.ds(tile_i * _BM, _BM), :]
    t = jnp.dot(tile, pTa_ref[...],
                preferred_element_type=jnp.float32)
    prev = tb_ref[pl.ds(1 - par, 1), :, :][0]
    acc_ref[...] = jnp.minimum(acc_ref[...],
                               jnp.min(prev, axis=0, keepdims=True))
    tb_ref[pl.ds(par, 1), :, :] = t[None]

    @pl.when(i == nm - 1)
    def _fin():
        out_ref[...] = jnp.sqrt(jnp.maximum(acc_ref[...] + psq_ref[...], 1e-12))


def kernel(features, W1, b1, W2, b2, memory_bank):
    B, L, C = features.shape
    N = B * L
    M, d2 = memory_bank.shape
    d1 = W1.shape[1]

    x = features.reshape(N, C)
    b1c = b1[:, None]
    b2c = b2[:, None]

    mpad = ((M + _BM - 1) // _BM) * _BM
    grid = (mpad // _BM + 1,)
    out = pl.pallas_call(
        _body,
        grid=grid,
        in_specs=[
            pl.BlockSpec((N, C), lambda i: (0, 0)),
            pl.BlockSpec((C, d1), lambda i: (0, 0)),
            pl.BlockSpec((d1, 1), lambda i: (0, 0)),
            pl.BlockSpec((d1, d2), lambda i: (0, 0)),
            pl.BlockSpec((d2, 1), lambda i: (0, 0)),
            pl.BlockSpec((M, d2), lambda i: (0, 0)),
        ],
        out_specs=pl.BlockSpec((1, N), lambda i: (0, 0)),
        out_shape=jax.ShapeDtypeStruct((1, N), jnp.float32),
        scratch_shapes=[
            pltpu.VMEM((mpad, _KA), jnp.bfloat16),
            pltpu.VMEM((_KA, N), jnp.bfloat16),
            pltpu.VMEM((1, N), jnp.float32),
            pltpu.VMEM((1, N), jnp.float32),
            pltpu.VMEM((2, _BM, N), jnp.float32),
        ],
    )(x, W1, b1c, W2, b2c, memory_bank)
    return out.reshape(B, L)


# speedup vs baseline: 1.6121x; 1.6121x over previous
import jax
import jax.numpy as jnp
from jax.experimental import pallas as pl
from jax.experimental.pallas import tpu as pltpu

_BM = 2048
_KA = 32


def _body(x_ref, w1_ref, b1_ref, w2_ref, b2_ref, bank_ref,
          out_ref, aug_ref, pTa_ref, psq_ref, acc_ref):
    i = pl.program_id(0)
    nm = pl.num_programs(0)
    n = x_ref.shape[0]

    @pl.when(i == 0)
    def _init():
        hT = jnp.maximum(
            jax.lax.dot_general(
                w1_ref[...], x_ref[...], (((0,), (1,)), ((), ())),
                preferred_element_type=jnp.float32) + b1_ref[...], 0.0)
        pT = jax.lax.dot_general(
            w2_ref[...], hT, (((0,), (0,)), ((), ())),
            preferred_element_type=jnp.float32) + b2_ref[...]
        psq_ref[...] = jnp.sum(pT * pT, axis=0, keepdims=True)
        d2 = pT.shape[0]
        pTa_ref[...] = jnp.concatenate(
            [-2.0 * pT,
             jnp.ones((1, n), jnp.float32),
             jnp.zeros((_KA - d2 - 1, n), jnp.float32)],
            axis=0).astype(jnp.bfloat16)
        bank = bank_ref[...]
        m = bank.shape[0]
        m_sq = jnp.sum(bank * bank, axis=1, keepdims=True)
        aug_ref[pl.ds(0, m), :] = jnp.concatenate(
            [bank, m_sq, jnp.zeros((m, _KA - d2 - 1), jnp.float32)],
            axis=1).astype(jnp.bfloat16)
        npad = aug_ref.shape[0] - m
        pad_rows = jnp.where(
            jax.lax.broadcasted_iota(jnp.int32, (npad, _KA), 1) == d2,
            jnp.float32(1e9), jnp.float32(0.0)).astype(jnp.bfloat16)
        aug_ref[pl.ds(m, npad), :] = pad_rows
        acc_ref[...] = jnp.full_like(acc_ref[...], jnp.inf)

    half = _BM // 2
    base = i * _BM
    pTa = pTa_ref[...]
    ta = jnp.dot(aug_ref[pl.ds(base, half), :], pTa,
                 preferred_element_type=jnp.float32)
    tb = jnp.dot(aug_ref[pl.ds(base + half, half), :], pTa,
                 preferred_element_type=jnp.float32)
    m = jnp.minimum(jnp.min(ta, axis=0, keepdims=True),
                    jnp.min(tb, axis=0, keepdims=True))
    acc_ref[...] = jnp.minimum(acc_ref[...], m)

    @pl.when(i == nm - 1)
    def _fin():
        out_ref[...] = jnp.sqrt(jnp.maximum(acc_ref[...] + psq_ref[...], 1e-12))


def kernel(features, W1, b1, W2, b2, memory_bank):
    B, L, C = features.shape
    N = B * L
    M, d2 = memory_bank.shape
    d1 = W1.shape[1]

    x = features.reshape(N, C)
    b1c = b1[:, None]
    b2c = b2[:, None]

    mpad = ((M + _BM - 1) // _BM) * _BM
    grid = (mpad // _BM,)
    out = pl.pallas_call(
        _body,
        grid=grid,
        in_specs=[
            pl.BlockSpec((N, C), lambda i: (0, 0)),
            pl.BlockSpec((C, d1), lambda i: (0, 0)),
            pl.BlockSpec((d1, 1), lambda i: (0, 0)),
            pl.BlockSpec((d1, d2), lambda i: (0, 0)),
            pl.BlockSpec((d2, 1), lambda i: (0, 0)),
            pl.BlockSpec((M, d2), lambda i: (0, 0)),
        ],
        out_specs=pl.BlockSpec((1, N), lambda i: (0, 0)),
        out_shape=jax.ShapeDtypeStruct((1, N), jnp.float32),
        scratch_shapes=[
            pltpu.VMEM((mpad, _KA), jnp.bfloat16),
            pltpu.VMEM((_KA, N), jnp.bfloat16),
            pltpu.VMEM((1, N), jnp.float32),
            pltpu.VMEM((1, N), jnp.float32),
        ],
    )(x, W1, b1c, W2, b2c, memory_bank)
    return out.reshape(B, L)


# speedup vs baseline: 1.6204x; 1.0051x over previous
import jax
import jax.numpy as jnp
from jax.experimental import pallas as pl
from jax.experimental.pallas import tpu as pltpu

_BM = 2560
_KA = 32


def _body(x_ref, w1_ref, b1_ref, w2_ref, b2_ref, bank_ref,
          out_ref, aug_ref, pTa_ref, psq_ref, acc_ref):
    i = pl.program_id(0)
    nm = pl.num_programs(0)
    n = x_ref.shape[0]

    @pl.when(i == 0)
    def _init():
        hT = jnp.maximum(
            jax.lax.dot_general(
                w1_ref[...], x_ref[...], (((0,), (1,)), ((), ())),
                preferred_element_type=jnp.float32) + b1_ref[...], 0.0)
        pT = jax.lax.dot_general(
            w2_ref[...], hT, (((0,), (0,)), ((), ())),
            preferred_element_type=jnp.float32) + b2_ref[...]
        psq_ref[...] = jnp.sum(pT * pT, axis=0, keepdims=True)
        d2 = pT.shape[0]
        pTa_ref[...] = jnp.concatenate(
            [-2.0 * pT,
             jnp.ones((1, n), jnp.float32),
             jnp.zeros((_KA - d2 - 1, n), jnp.float32)],
            axis=0).astype(jnp.bfloat16)
        bank = bank_ref[...]
        m = bank.shape[0]
        m_sq = jnp.sum(bank * bank, axis=1, keepdims=True)
        aug_ref[pl.ds(0, m), :] = jnp.concatenate(
            [bank, m_sq, jnp.zeros((m, _KA - d2 - 1), jnp.float32)],
            axis=1).astype(jnp.bfloat16)
        npad = aug_ref.shape[0] - m
        pad_rows = jnp.where(
            jax.lax.broadcasted_iota(jnp.int32, (npad, _KA), 1) == d2,
            jnp.float32(1e9), jnp.float32(0.0)).astype(jnp.bfloat16)
        aug_ref[pl.ds(m, npad), :] = pad_rows
        acc_ref[...] = jnp.full_like(acc_ref[...], jnp.inf)

    half = _BM // 2
    base = i * _BM
    pTa = pTa_ref[...]
    ta = jnp.dot(aug_ref[pl.ds(base, half), :], pTa,
                 preferred_element_type=jnp.float32)
    tb = jnp.dot(aug_ref[pl.ds(base + half, half), :], pTa,
                 preferred_element_type=jnp.float32)
    m = jnp.minimum(jnp.min(ta, axis=0, keepdims=True),
                    jnp.min(tb, axis=0, keepdims=True))
    acc_ref[...] = jnp.minimum(acc_ref[...], m)

    @pl.when(i == nm - 1)
    def _fin():
        out_ref[...] = jnp.sqrt(jnp.maximum(acc_ref[...] + psq_ref[...], 1e-12))


def kernel(features, W1, b1, W2, b2, memory_bank):
    B, L, C = features.shape
    N = B * L
    M, d2 = memory_bank.shape
    d1 = W1.shape[1]

    x = features.reshape(N, C)
    b1c = b1[:, None]
    b2c = b2[:, None]

    mpad = ((M + _BM - 1) // _BM) * _BM
    grid = (mpad // _BM,)
    out = pl.pallas_call(
        _body,
        grid=grid,
        in_specs=[
            pl.BlockSpec((N, C), lambda i: (0, 0)),
            pl.BlockSpec((C, d1), lambda i: (0, 0)),
            pl.BlockSpec((d1, 1), lambda i: (0, 0)),
            pl.BlockSpec((d1, d2), lambda i: (0, 0)),
            pl.BlockSpec((d2, 1), lambda i: (0, 0)),
            pl.BlockSpec((M, d2), lambda i: (0, 0)),
        ],
        out_specs=pl.BlockSpec((1, N), lambda i: (0, 0)),
        out_shape=jax.ShapeDtypeStruct((1, N), jnp.float32),
        scratch_shapes=[
            pltpu.VMEM((mpad, _KA), jnp.bfloat16),
            pltpu.VMEM((_KA, N), jnp.bfloat16),
            pltpu.VMEM((1, N), jnp.float32),
            pltpu.VMEM((1, N), jnp.float32),
        ],
    )(x, W1, b1c, W2, b2c, memory_bank)
    return out.reshape(B, L)
